# Initial kernel scaffold; baseline (speedup 1.0000x reference)
import jax
import jax.numpy as jnp
from jax.experimental import pallas as pl


def kernel(feature_input, edge_index, batch_index, number_of_trajectories, stats, gcn_W, gcn_b, W_ih, W_hh, b_ih, b_hh, W1, b1, W2, b2, W3, b3, W4, b4):
    raise NotImplementedError("write your pallas kernel here")


# speedup vs baseline: 15.0792x; 15.0792x over previous
import functools

import jax
import jax.numpy as jnp
from jax import lax
from jax.experimental import pallas as pl
from jax.experimental.pallas import tpu as pltpu
from jax.experimental.pallas import tpu_sc as plsc

NUM_NODES = 30
NUM_IN = 2
GCN_OUT = 64
HID = 512
T = 128
S_IN = 5
S_OUT = 3
N_TOTAL = T * NUM_NODES
N_EDGES = 32768

NN_PAD = 11776
SLICE = NN_PAD // 16
E_CORE = 3 * N_EDGES
E_TILE = E_CORE // 16
CHUNK = 128
NCHUNK = E_TILE // CHUNK
DUMMY_DST = 11520
NODE_PAD = 32


TBL = 32776


def _sc_body(src_hbm, dst_hbm, x0_hbm, x1_hbm, tbl_hbm, out0_hbm, out1_hbm,
             src_v, dst_v, x0_v, x1_v, dinv_v, vals0_v, vals1_v,
             degs_v, dinvs_v, s0_v, s1_v, ones_v, table_v,
             deg_sh, dinv_sh, out0_sh, out1_sh):
    c = lax.axis_index("c")
    s = lax.axis_index("s")
    base = s * SLICE

    pltpu.sync_copy(src_hbm.at[c, pl.ds(s * NCHUNK, NCHUNK)], src_v)
    pltpu.sync_copy(dst_hbm.at[c, pl.ds(s * NCHUNK, NCHUNK)], dst_v)
    pltpu.sync_copy(x0_hbm.at[pl.ds(c * NN_PAD, NN_PAD)], x0_v)
    pltpu.sync_copy(x1_hbm.at[pl.ds(c * NN_PAD, NN_PAD)], x1_v)
    pltpu.sync_copy(tbl_hbm, table_v)

    for i in range(CHUNK // 16):
        ones_v[pl.ds(i * 16, 16)] = jnp.full((16,), 1.0, jnp.float32)
    for i in range(SLICE // 16):
        degs_v[pl.ds(i * 16, 16)] = jnp.full((16,), 0.0, jnp.float32)
    pltpu.sync_copy(degs_v, deg_sh.at[pl.ds(base, SLICE)])
    plsc.subcore_barrier()

    def deg_step(j, carry):
        pltpu.sync_copy(ones_v, deg_sh.at[dst_v.at[j]], add=True)
        return carry
    lax.fori_loop(0, NCHUNK, deg_step, 0)
    plsc.subcore_barrier()

    pltpu.sync_copy(deg_sh.at[pl.ds(base, SLICE)], degs_v)
    for i in range(SLICE // 16):
        d = degs_v[pl.ds(i * 16, 16)] + 1.0
        y = plsc.load_gather(table_v, [d.astype(jnp.int32)])
        dinvs_v[pl.ds(i * 16, 16)] = y
        y2 = y * y
        s0_v[pl.ds(i * 16, 16)] = x0_v[pl.ds(base + i * 16, 16)] * y2
        s1_v[pl.ds(i * 16, 16)] = x1_v[pl.ds(base + i * 16, 16)] * y2
    pltpu.sync_copy(dinvs_v, dinv_sh.at[pl.ds(base, SLICE)])
    pltpu.sync_copy(s0_v, out0_sh.at[pl.ds(base, SLICE)])
    pltpu.sync_copy(s1_v, out1_sh.at[pl.ds(base, SLICE)])
    plsc.subcore_barrier()

    pltpu.sync_copy(dinv_sh, dinv_v)

    def edge_step(j, carry):
        for k in range(CHUNK // 16):
            src = src_v[j, pl.ds(k * 16, 16)]
            dst = dst_v[j, pl.ds(k * 16, 16)]
            coef = (plsc.load_gather(dinv_v, [src])
                    * plsc.load_gather(dinv_v, [dst]))
            vals0_v[pl.ds(k * 16, 16)] = plsc.load_gather(x0_v, [src]) * coef
            vals1_v[pl.ds(k * 16, 16)] = plsc.load_gather(x1_v, [src]) * coef
        pltpu.sync_copy(vals0_v, out0_sh.at[dst_v.at[j]], add=True)
        pltpu.sync_copy(vals1_v, out1_sh.at[dst_v.at[j]], add=True)
        return carry
    lax.fori_loop(0, NCHUNK, edge_step, 0)
    plsc.subcore_barrier()

    pltpu.sync_copy(out0_sh.at[pl.ds(base, SLICE)], s0_v)
    pltpu.sync_copy(s0_v, out0_hbm.at[pl.ds(c * NN_PAD + base, SLICE)])
    pltpu.sync_copy(out1_sh.at[pl.ds(base, SLICE)], s1_v)
    pltpu.sync_copy(s1_v, out1_hbm.at[pl.ds(c * NN_PAD + base, SLICE)])


@functools.cache
def _sc_scatter_kernel():
    return functools.partial(
        pl.kernel,
        out_type=[jax.ShapeDtypeStruct((2 * NN_PAD,), jnp.float32),
                  jax.ShapeDtypeStruct((2 * NN_PAD,), jnp.float32)],
        mesh=plsc.VectorSubcoreMesh(core_axis_name="c", subcore_axis_name="s",
                                    num_cores=2, num_subcores=16),
        compiler_params=pltpu.CompilerParams(needs_layout_passes=False),
        scratch_types=[
        pltpu.VMEM((NCHUNK, CHUNK), jnp.int32),
        pltpu.VMEM((NCHUNK, CHUNK), jnp.int32),
        pltpu.VMEM((NN_PAD,), jnp.float32),
        pltpu.VMEM((NN_PAD,), jnp.float32),
        pltpu.VMEM((NN_PAD,), jnp.float32),
        pltpu.VMEM((CHUNK,), jnp.float32),
        pltpu.VMEM((CHUNK,), jnp.float32),
        pltpu.VMEM((SLICE,), jnp.float32),
        pltpu.VMEM((SLICE,), jnp.float32),
        pltpu.VMEM((SLICE,), jnp.float32),
        pltpu.VMEM((SLICE,), jnp.float32),
        pltpu.VMEM((CHUNK,), jnp.float32),
        pltpu.VMEM((TBL,), jnp.float32),
        pltpu.VMEM_SHARED((NN_PAD,), jnp.float32),
        pltpu.VMEM_SHARED((NN_PAD,), jnp.float32),
        pltpu.VMEM_SHARED((NN_PAD,), jnp.float32),
        pltpu.VMEM_SHARED((NN_PAD,), jnp.float32),
        ],
    )(_sc_body)


def _fold_body(p_ref, w_ref, u_ref):
    u_ref[...] = jnp.dot(p_ref[...], w_ref[...],
                         preferred_element_type=jnp.float32)


def _fold_u(P, W_ih):
    n_blk = 8
    blk = (4 * HID) // n_blk
    return pl.pallas_call(
        _fold_body,
        grid=(n_blk,),
        in_specs=[
            pl.BlockSpec((72, GCN_OUT * NUM_NODES), lambda n: (0, 0)),
            pl.BlockSpec((GCN_OUT * NUM_NODES, blk), lambda n: (0, n)),
        ],
        out_specs=pl.BlockSpec((72, blk), lambda n: (0, n)),
        out_shape=jax.ShapeDtypeStruct((72, 4 * HID), jnp.float32),
    )(P, W_ih)


def _seq_body(o0_ref, o1_ref, u_ref, whh_ref, bsum_ref,
              w1_ref, b1_ref, w2_ref, b2_ref, w3_ref, b3_ref,
              w4_ref, b4_ref, sx_ref, sy_ref, stats_ref, preds_ref):
    U0 = u_ref[0:32, :]
    U1 = u_ref[32:64, :]
    bvec = u_ref[64:65, :]
    bias = bvec + bsum_ref[...]
    std0 = stats_ref[0:1, 0:1]
    std1 = stats_ref[0:1, 1:2]
    mean0 = stats_ref[1:2, 0:1]
    mean1 = stats_ref[1:2, 1:2]

    h = jnp.zeros((T, HID), jnp.float32)
    c = jnp.zeros((T, HID), jnp.float32)
    p = None
    for step in range(S_IN - 1 + S_OUT):
        if step < S_IN:
            m0 = o0_ref[step]
            m1 = o1_ref[step]
        else:
            pxs = jnp.dot(p, sx_ref[...], preferred_element_type=jnp.float32)
            pys = jnp.dot(p, sy_ref[...], preferred_element_type=jnp.float32)
            fx = pxs * std0 + mean0
            fy = pys * std1 + mean1
            col = lax.broadcasted_iota(jnp.int32, (T, NODE_PAD), 1)
            exn = jnp.where((fx > 0.04) & (fy > 0.04) & (col < NUM_NODES),
                            1.0, 0.0)
            dx = fx[:, :, None] - fx[:, None, :]
            dy = fy[:, :, None] - fy[:, None, :]
            d2 = dx * dx + dy * dy
            cond = jnp.where((d2 > 0.0) & (d2 < 100.0), 1.0, 0.0)
            ep = exn[:, :, None] * exn[:, None, :]
            r = lax.broadcasted_iota(jnp.int32, (T, NODE_PAD, NODE_PAD), 1)
            q = lax.broadcasted_iota(jnp.int32, (T, NODE_PAD, NODE_PAD), 2)
            eye = r == q
            A = jnp.where(eye, 1.0, ep * cond)
            deg = jnp.sum(A, axis=-1)
            dinv = lax.rsqrt(jnp.maximum(deg, 1e-12))
            An = A * (dinv[:, :, None] * dinv[:, None, :])
            m0 = jnp.sum(An * pxs[:, None, :], axis=-1)
            m1 = jnp.sum(An * pys[:, None, :], axis=-1)
        gates = (jnp.dot(m0, U0, preferred_element_type=jnp.float32)
                 + jnp.dot(m1, U1, preferred_element_type=jnp.float32)
                 + jnp.dot(h, whh_ref[...], preferred_element_type=jnp.float32)
                 + bias)
        i_g = jax.nn.sigmoid(gates[:, 0:HID])
        f_g = jax.nn.sigmoid(gates[:, HID:2 * HID])
        g_g = jnp.tanh(gates[:, 2 * HID:3 * HID])
        o_g = jax.nn.sigmoid(gates[:, 3 * HID:4 * HID])
        c = f_g * c + i_g * g_g
        h = o_g * jnp.tanh(c)
        m = jax.nn.relu(jnp.dot(h, w1_ref[...],
                                preferred_element_type=jnp.float32)
                        + b1_ref[...])
        m = jax.nn.relu(jnp.dot(m, w2_ref[...],
                                preferred_element_type=jnp.float32)
                        + b2_ref[...])
        m = jax.nn.relu(jnp.dot(m, w3_ref[...],
                                preferred_element_type=jnp.float32)
                        + b3_ref[...])
        p = jnp.dot(m, w4_ref[...],
                    preferred_element_type=jnp.float32) + b4_ref[...]
        preds_ref[step] = p


def _seq_run(o0p, o1p, U, W_hh, bsum, W1, b1, W2, b2, W3, b3, W4, b4,
             Sx, Sy, stats):
    return pl.pallas_call(
        _seq_body,
        out_shape=jax.ShapeDtypeStruct((7, T, NUM_IN * NUM_NODES),
                                       jnp.float32),
    )(o0p, o1p, U, W_hh, bsum, W1, b1, W2, b2, W3, b3, W4, b4, Sx, Sy, stats)


def kernel(feature_input, edge_index, batch_index, number_of_trajectories,
           stats, gcn_W, gcn_b, W_ih, W_hh, b_ih, b_hh,
           W1, b1, W2, b2, W3, b3, W4, b4):
    ei = edge_index.astype(jnp.int32)
    src0 = jnp.concatenate([ei[0, 0], ei[1, 0] + N_TOTAL, ei[2, 0] + 2 * N_TOTAL])
    dst0 = jnp.concatenate([ei[0, 1], ei[1, 1] + N_TOTAL, ei[2, 1] + 2 * N_TOTAL])
    dummy_src = jnp.zeros((N_EDGES,), jnp.int32)
    dummy_dst = jnp.full((N_EDGES,), DUMMY_DST, jnp.int32)
    src1 = jnp.concatenate([ei[3, 0], ei[4, 0] + N_TOTAL, dummy_src])
    dst1 = jnp.concatenate([ei[3, 1], ei[4, 1] + N_TOTAL, dummy_dst])
    src_e = jnp.stack([src0, src1]).reshape(2, E_CORE // CHUNK, CHUNK)
    dst_e = jnp.stack([dst0, dst1]).reshape(2, E_CORE // CHUNK, CHUNK)

    xf = feature_input.reshape(S_IN * N_TOTAL, NUM_IN)
    pad0 = NN_PAD - 3 * N_TOTAL
    pad1 = NN_PAD - 2 * N_TOTAL
    x0_in = jnp.concatenate([
        jnp.pad(xf[:3 * N_TOTAL, 0], (0, pad0)),
        jnp.pad(xf[3 * N_TOTAL:, 0], (0, pad1)),
    ])
    x1_in = jnp.concatenate([
        jnp.pad(xf[:3 * N_TOTAL, 1], (0, pad0)),
        jnp.pad(xf[3 * N_TOTAL:, 1], (0, pad1)),
    ])

    rsqrt_tbl = lax.rsqrt(jnp.maximum(
        jnp.arange(TBL, dtype=jnp.float32), 1.0))
    out0, out1 = _sc_scatter_kernel()(src_e, dst_e, x0_in, x1_in, rsqrt_tbl)
    o0 = jnp.concatenate([out0[:3 * N_TOTAL],
                          out0[NN_PAD:NN_PAD + 2 * N_TOTAL]])
    o1 = jnp.concatenate([out1[:3 * N_TOTAL],
                          out1[NN_PAD:NN_PAD + 2 * N_TOTAL]])
    o0p = jnp.pad(o0.reshape(S_IN, T, NUM_NODES), ((0, 0), (0, 0), (0, 2)))
    o1p = jnp.pad(o1.reshape(S_IN, T, NUM_NODES), ((0, 0), (0, 0), (0, 2)))

    K = GCN_OUT * NUM_NODES
    eye30 = jnp.eye(NUM_NODES, dtype=jnp.float32)
    P0 = jnp.kron(eye30, gcn_W[0:1, :])
    P1 = jnp.kron(eye30, gcn_W[1:2, :])
    bb = jnp.tile(gcn_b, NUM_NODES)[None, :]
    zrow2 = jnp.zeros((2, K), jnp.float32)
    zrow7 = jnp.zeros((7, K), jnp.float32)
    P = jnp.concatenate([P0, zrow2, P1, zrow2, bb, zrow7])
    U = _fold_u(P, W_ih)

    bsum = (b_ih + b_hh)[None, :]
    k60 = jnp.arange(NUM_IN * NUM_NODES)[:, None]
    n32 = jnp.arange(NODE_PAD)[None, :]
    Sx = ((k60 == 2 * n32) & (n32 < NUM_NODES)).astype(jnp.float32)
    Sy = ((k60 == 2 * n32 + 1) & (n32 < NUM_NODES)).astype(jnp.float32)

    preds = _seq_run(o0p, o1p, U, W_hh, bsum,
                     W1, b1[None, :], W2, b2[None, :], W3, b3[None, :],
                     W4, b4[None, :], Sx, Sy, stats)

    enc = jnp.concatenate([
        feature_input[0][None],
        preds[:S_IN - 1].reshape(S_IN - 1, N_TOTAL, NUM_IN),
    ])
    dec = preds[S_IN - 1:].reshape(S_OUT, N_TOTAL, NUM_IN)
    return enc, dec


# speedup vs baseline: 26.6087x; 1.7646x over previous
import functools

import jax
import jax.numpy as jnp
from jax import lax
from jax.experimental import pallas as pl
from jax.experimental.pallas import tpu as pltpu
from jax.experimental.pallas import tpu_sc as plsc

NUM_NODES = 30
NUM_IN = 2
GCN_OUT = 64
HID = 512
T = 128
S_IN = 5
S_OUT = 3
N_TOTAL = T * NUM_NODES
N_EDGES = 32768

NN_PAD = 11776
SLICE = NN_PAD // 16
E_CORE = 3 * N_EDGES
E_TILE = E_CORE // 16
CHUNK = 128
NCHUNK = E_TILE // CHUNK
DUMMY_DST = 11520
NODE_PAD = 32


TBL = 32776


def _sc_body(src_hbm, dst_hbm, x0_hbm, x1_hbm, tbl_hbm, out0_hbm, out1_hbm,
             src_v, dst_v, x0_v, x1_v, dinv_v, vals0_v, vals1_v,
             degs_v, dinvs_v, s0_v, s1_v, ones_v, table_v,
             deg_sh, dinv_sh, out0_sh, out1_sh):
    c = lax.axis_index("c")
    s = lax.axis_index("s")
    base = s * SLICE
    nch = jnp.where(c == 0, NCHUNK, (2 * N_EDGES) // (16 * CHUNK))

    pltpu.sync_copy(src_hbm.at[c, pl.ds(s * NCHUNK, NCHUNK)], src_v)
    pltpu.sync_copy(dst_hbm.at[c, pl.ds(s * NCHUNK, NCHUNK)], dst_v)
    pltpu.sync_copy(x0_hbm.at[pl.ds(c * NN_PAD, NN_PAD)], x0_v)
    pltpu.sync_copy(x1_hbm.at[pl.ds(c * NN_PAD, NN_PAD)], x1_v)
    pltpu.sync_copy(tbl_hbm, table_v)

    for i in range(CHUNK // 16):
        ones_v[pl.ds(i * 16, 16)] = jnp.full((16,), 1.0, jnp.float32)
    for i in range(SLICE // 16):
        degs_v[pl.ds(i * 16, 16)] = jnp.full((16,), 0.0, jnp.float32)
    pltpu.sync_copy(degs_v, deg_sh.at[pl.ds(base, SLICE)])
    plsc.subcore_barrier()

    def deg_step(j, carry):
        pltpu.sync_copy(ones_v, deg_sh.at[dst_v.at[j]], add=True)
        return carry
    lax.fori_loop(0, nch, deg_step, 0)
    plsc.subcore_barrier()

    pltpu.sync_copy(deg_sh.at[pl.ds(base, SLICE)], degs_v)
    for i in range(SLICE // 16):
        d = degs_v[pl.ds(i * 16, 16)] + 1.0
        y = plsc.load_gather(table_v, [d.astype(jnp.int32)])
        dinvs_v[pl.ds(i * 16, 16)] = y
        y2 = y * y
        s0_v[pl.ds(i * 16, 16)] = x0_v[pl.ds(base + i * 16, 16)] * y2
        s1_v[pl.ds(i * 16, 16)] = x1_v[pl.ds(base + i * 16, 16)] * y2
    pltpu.sync_copy(dinvs_v, dinv_sh.at[pl.ds(base, SLICE)])
    pltpu.sync_copy(s0_v, out0_sh.at[pl.ds(base, SLICE)])
    pltpu.sync_copy(s1_v, out1_sh.at[pl.ds(base, SLICE)])
    plsc.subcore_barrier()

    pltpu.sync_copy(dinv_sh, dinv_v)

    def edge_step(j, carry):
        for k in range(CHUNK // 16):
            src = src_v[j, pl.ds(k * 16, 16)]
            dst = dst_v[j, pl.ds(k * 16, 16)]
            coef = (plsc.load_gather(dinv_v, [src])
                    * plsc.load_gather(dinv_v, [dst]))
            vals0_v[pl.ds(k * 16, 16)] = plsc.load_gather(x0_v, [src]) * coef
            vals1_v[pl.ds(k * 16, 16)] = plsc.load_gather(x1_v, [src]) * coef
        pltpu.sync_copy(vals0_v, out0_sh.at[dst_v.at[j]], add=True)
        pltpu.sync_copy(vals1_v, out1_sh.at[dst_v.at[j]], add=True)
        return carry
    lax.fori_loop(0, nch, edge_step, 0)
    plsc.subcore_barrier()

    pltpu.sync_copy(out0_sh.at[pl.ds(base, SLICE)], s0_v)
    pltpu.sync_copy(s0_v, out0_hbm.at[pl.ds(c * NN_PAD + base, SLICE)])
    pltpu.sync_copy(out1_sh.at[pl.ds(base, SLICE)], s1_v)
    pltpu.sync_copy(s1_v, out1_hbm.at[pl.ds(c * NN_PAD + base, SLICE)])


@functools.cache
def _sc_scatter_kernel():
    return functools.partial(
        pl.kernel,
        out_type=[jax.ShapeDtypeStruct((2 * NN_PAD,), jnp.float32),
                  jax.ShapeDtypeStruct((2 * NN_PAD,), jnp.float32)],
        mesh=plsc.VectorSubcoreMesh(core_axis_name="c", subcore_axis_name="s",
                                    num_cores=2, num_subcores=16),
        compiler_params=pltpu.CompilerParams(needs_layout_passes=False),
        scratch_types=[
        pltpu.VMEM((NCHUNK, CHUNK), jnp.int32),
        pltpu.VMEM((NCHUNK, CHUNK), jnp.int32),
        pltpu.VMEM((NN_PAD,), jnp.float32),
        pltpu.VMEM((NN_PAD,), jnp.float32),
        pltpu.VMEM((NN_PAD,), jnp.float32),
        pltpu.VMEM((CHUNK,), jnp.float32),
        pltpu.VMEM((CHUNK,), jnp.float32),
        pltpu.VMEM((SLICE,), jnp.float32),
        pltpu.VMEM((SLICE,), jnp.float32),
        pltpu.VMEM((SLICE,), jnp.float32),
        pltpu.VMEM((SLICE,), jnp.float32),
        pltpu.VMEM((CHUNK,), jnp.float32),
        pltpu.VMEM((TBL,), jnp.float32),
        pltpu.VMEM_SHARED((NN_PAD,), jnp.float32),
        pltpu.VMEM_SHARED((NN_PAD,), jnp.float32),
        pltpu.VMEM_SHARED((NN_PAD,), jnp.float32),
        pltpu.VMEM_SHARED((NN_PAD,), jnp.float32),
        ],
    )(_sc_body)


def _fold_body(p_ref, w_ref, u_ref):
    u_ref[...] = jnp.dot(p_ref[...], w_ref[...],
                         preferred_element_type=jnp.float32)


def _fold_u(P, W_ih):
    n_blk = 8
    blk = (4 * HID) // n_blk
    return pl.pallas_call(
        _fold_body,
        grid=(n_blk,),
        in_specs=[
            pl.BlockSpec((72, GCN_OUT * NUM_NODES), lambda n: (0, 0)),
            pl.BlockSpec((GCN_OUT * NUM_NODES, blk), lambda n: (0, n)),
        ],
        out_specs=pl.BlockSpec((72, blk), lambda n: (0, n)),
        out_shape=jax.ShapeDtypeStruct((72, 4 * HID), jnp.float32),
    )(P, W_ih)


def _seq_body(o0_ref, o1_ref, u_ref, whh_ref, bsum_ref,
              w1_ref, b1_ref, w2_ref, b2_ref, w3_ref, b3_ref,
              w4_ref, b4_ref, sx_ref, sy_ref, stats_ref, preds_ref):
    U0 = u_ref[0:32, :]
    U1 = u_ref[32:64, :]
    bvec = u_ref[64:65, :]
    bias = bvec + bsum_ref[...]
    std0 = stats_ref[0:1, 0:1]
    std1 = stats_ref[0:1, 1:2]
    mean0 = stats_ref[1:2, 0:1]
    mean1 = stats_ref[1:2, 1:2]

    h = jnp.zeros((T, HID), jnp.float32)
    c = jnp.zeros((T, HID), jnp.float32)
    p = None
    for step in range(S_IN - 1 + S_OUT):
        if step < S_IN:
            m0 = o0_ref[step]
            m1 = o1_ref[step]
        else:
            pxs = jnp.dot(p, sx_ref[...], preferred_element_type=jnp.float32)
            pys = jnp.dot(p, sy_ref[...], preferred_element_type=jnp.float32)
            fx = pxs * std0 + mean0
            fy = pys * std1 + mean1
            col = lax.broadcasted_iota(jnp.int32, (T, NODE_PAD), 1)
            exn = jnp.where((fx > 0.04) & (fy > 0.04) & (col < NUM_NODES),
                            1.0, 0.0)
            dx = fx[:, :, None] - fx[:, None, :]
            dy = fy[:, :, None] - fy[:, None, :]
            d2 = dx * dx + dy * dy
            cond = jnp.where((d2 > 0.0) & (d2 < 100.0), 1.0, 0.0)
            ep = exn[:, :, None] * exn[:, None, :]
            r = lax.broadcasted_iota(jnp.int32, (T, NODE_PAD, NODE_PAD), 1)
            q = lax.broadcasted_iota(jnp.int32, (T, NODE_PAD, NODE_PAD), 2)
            eye = r == q
            A = jnp.where(eye, 1.0, ep * cond)
            deg = jnp.sum(A, axis=-1)
            dinv = lax.rsqrt(jnp.maximum(deg, 1e-12))
            An = A * (dinv[:, :, None] * dinv[:, None, :])
            m0 = jnp.sum(An * pxs[:, None, :], axis=-1)
            m1 = jnp.sum(An * pys[:, None, :], axis=-1)
        gates = (jnp.dot(m0, U0, preferred_element_type=jnp.float32)
                 + jnp.dot(m1, U1, preferred_element_type=jnp.float32)
                 + jnp.dot(h, whh_ref[...], preferred_element_type=jnp.float32)
                 + bias)
        i_g = jax.nn.sigmoid(gates[:, 0:HID])
        f_g = jax.nn.sigmoid(gates[:, HID:2 * HID])
        g_g = jnp.tanh(gates[:, 2 * HID:3 * HID])
        o_g = jax.nn.sigmoid(gates[:, 3 * HID:4 * HID])
        c = f_g * c + i_g * g_g
        h = o_g * jnp.tanh(c)
        m = jax.nn.relu(jnp.dot(h, w1_ref[...],
                                preferred_element_type=jnp.float32)
                        + b1_ref[...])
        m = jax.nn.relu(jnp.dot(m, w2_ref[...],
                                preferred_element_type=jnp.float32)
                        + b2_ref[...])
        m = jax.nn.relu(jnp.dot(m, w3_ref[...],
                                preferred_element_type=jnp.float32)
                        + b3_ref[...])
        p = jnp.dot(m, w4_ref[...],
                    preferred_element_type=jnp.float32) + b4_ref[...]
        preds_ref[step] = p


def _seq_run(o0p, o1p, U, W_hh, bsum, W1, b1, W2, b2, W3, b3, W4, b4,
             Sx, Sy, stats):
    return pl.pallas_call(
        _seq_body,
        out_shape=jax.ShapeDtypeStruct((7, T, NUM_IN * NUM_NODES),
                                       jnp.float32),
    )(o0p, o1p, U, W_hh, bsum, W1, b1, W2, b2, W3, b3, W4, b4, Sx, Sy, stats)


def kernel(feature_input, edge_index, batch_index, number_of_trajectories,
           stats, gcn_W, gcn_b, W_ih, W_hh, b_ih, b_hh,
           W1, b1, W2, b2, W3, b3, W4, b4):
    ei = edge_index.astype(jnp.int32)
    src0 = jnp.concatenate([ei[0, 0], ei[1, 0] + N_TOTAL, ei[2, 0] + 2 * N_TOTAL])
    dst0 = jnp.concatenate([ei[0, 1], ei[1, 1] + N_TOTAL, ei[2, 1] + 2 * N_TOTAL])
    def _tile_pad(arr, fill):
        real = arr.reshape(16, 4096)
        dummy = jnp.full((16, E_TILE - 4096), fill, jnp.int32)
        return jnp.concatenate([real, dummy], axis=1).reshape(-1)
    src1 = _tile_pad(jnp.concatenate([ei[3, 0], ei[4, 0] + N_TOTAL]), 0)
    dst1 = _tile_pad(jnp.concatenate([ei[3, 1], ei[4, 1] + N_TOTAL]),
                     DUMMY_DST)
    src_e = jnp.stack([src0, src1]).reshape(2, E_CORE // CHUNK, CHUNK)
    dst_e = jnp.stack([dst0, dst1]).reshape(2, E_CORE // CHUNK, CHUNK)

    xf = feature_input.reshape(S_IN * N_TOTAL, NUM_IN)
    pad0 = NN_PAD - 3 * N_TOTAL
    pad1 = NN_PAD - 2 * N_TOTAL
    x0_in = jnp.concatenate([
        jnp.pad(xf[:3 * N_TOTAL, 0], (0, pad0)),
        jnp.pad(xf[3 * N_TOTAL:, 0], (0, pad1)),
    ])
    x1_in = jnp.concatenate([
        jnp.pad(xf[:3 * N_TOTAL, 1], (0, pad0)),
        jnp.pad(xf[3 * N_TOTAL:, 1], (0, pad1)),
    ])

    rsqrt_tbl = lax.rsqrt(jnp.maximum(
        jnp.arange(TBL, dtype=jnp.float32), 1.0))
    out0, out1 = _sc_scatter_kernel()(src_e, dst_e, x0_in, x1_in, rsqrt_tbl)
    o0 = jnp.concatenate([out0[:3 * N_TOTAL],
                          out0[NN_PAD:NN_PAD + 2 * N_TOTAL]])
    o1 = jnp.concatenate([out1[:3 * N_TOTAL],
                          out1[NN_PAD:NN_PAD + 2 * N_TOTAL]])
    o0p = jnp.pad(o0.reshape(S_IN, T, NUM_NODES), ((0, 0), (0, 0), (0, 2)))
    o1p = jnp.pad(o1.reshape(S_IN, T, NUM_NODES), ((0, 0), (0, 0), (0, 2)))

    K = GCN_OUT * NUM_NODES
    eye30 = jnp.eye(NUM_NODES, dtype=jnp.float32)
    P0 = jnp.kron(eye30, gcn_W[0:1, :])
    P1 = jnp.kron(eye30, gcn_W[1:2, :])
    bb = jnp.tile(gcn_b, NUM_NODES)[None, :]
    zrow2 = jnp.zeros((2, K), jnp.float32)
    zrow7 = jnp.zeros((7, K), jnp.float32)
    P = jnp.concatenate([P0, zrow2, P1, zrow2, bb, zrow7])
    U = _fold_u(P, W_ih)

    bsum = (b_ih + b_hh)[None, :]
    k60 = jnp.arange(NUM_IN * NUM_NODES)[:, None]
    n32 = jnp.arange(NODE_PAD)[None, :]
    Sx = ((k60 == 2 * n32) & (n32 < NUM_NODES)).astype(jnp.float32)
    Sy = ((k60 == 2 * n32 + 1) & (n32 < NUM_NODES)).astype(jnp.float32)

    preds = _seq_run(o0p, o1p, U, W_hh, bsum,
                     W1, b1[None, :], W2, b2[None, :], W3, b3[None, :],
                     W4, b4[None, :], Sx, Sy, stats)

    enc = jnp.concatenate([
        feature_input[0][None],
        preds[:S_IN - 1].reshape(S_IN - 1, N_TOTAL, NUM_IN),
    ])
    dec = preds[S_IN - 1:].reshape(S_OUT, N_TOTAL, NUM_IN)
    return enc, dec


# speedup vs baseline: 27.7025x; 1.0411x over previous
import functools

import jax
import jax.numpy as jnp
from jax import lax
from jax.experimental import pallas as pl
from jax.experimental.pallas import tpu as pltpu
from jax.experimental.pallas import tpu_sc as plsc

NUM_NODES = 30
NUM_IN = 2
GCN_OUT = 64
HID = 512
T = 128
S_IN = 5
S_OUT = 3
N_TOTAL = T * NUM_NODES
N_EDGES = 32768

NN_PAD = 11776
SLICE = NN_PAD // 16
E_CORE = 3 * N_EDGES
E_TILE = E_CORE // 16
CHUNK = 128
NCHUNK = E_TILE // CHUNK
DUMMY_DST = 11520
NODE_PAD = 32


TBL = 32776


def _sc_body(src_hbm, dst_hbm, x0_hbm, x1_hbm, tbl_hbm, out0_hbm, out1_hbm,
             src_v, dst_v, x0_v, x1_v, vals0_v, vals1_v,
             degs_v, dinvs_v, s0_v, s1_v, ones_v, table_v,
             deg_sh, xs0_sh, xs1_sh, out0_sh, out1_sh):
    c = lax.axis_index("c")
    s = lax.axis_index("s")
    base = s * SLICE
    nch = jnp.where(c == 0, NCHUNK, (2 * N_EDGES) // (16 * CHUNK))

    pltpu.sync_copy(src_hbm.at[c, pl.ds(s * NCHUNK, NCHUNK)], src_v)
    pltpu.sync_copy(dst_hbm.at[c, pl.ds(s * NCHUNK, NCHUNK)], dst_v)
    pltpu.sync_copy(x0_hbm.at[pl.ds(c * NN_PAD + base, SLICE)], s0_v)
    pltpu.sync_copy(x1_hbm.at[pl.ds(c * NN_PAD + base, SLICE)], s1_v)
    pltpu.sync_copy(tbl_hbm, table_v)

    for i in range(CHUNK // 16):
        ones_v[pl.ds(i * 16, 16)] = jnp.full((16,), 1.0, jnp.float32)
    for i in range(SLICE // 16):
        degs_v[pl.ds(i * 16, 16)] = jnp.full((16,), 0.0, jnp.float32)
    pltpu.sync_copy(degs_v, deg_sh.at[pl.ds(base, SLICE)])
    plsc.subcore_barrier()

    def deg_step(j, carry):
        pltpu.sync_copy(ones_v, deg_sh.at[dst_v.at[j]], add=True)
        return carry
    lax.fori_loop(0, nch, deg_step, 0)
    plsc.subcore_barrier()

    pltpu.sync_copy(deg_sh.at[pl.ds(base, SLICE)], degs_v)
    for i in range(SLICE // 16):
        d = degs_v[pl.ds(i * 16, 16)] + 1.0
        y = plsc.load_gather(table_v, [d.astype(jnp.int32)])
        dinvs_v[pl.ds(i * 16, 16)] = y
        s0_v[pl.ds(i * 16, 16)] = s0_v[pl.ds(i * 16, 16)] * y
        s1_v[pl.ds(i * 16, 16)] = s1_v[pl.ds(i * 16, 16)] * y
    pltpu.sync_copy(s0_v, xs0_sh.at[pl.ds(base, SLICE)])
    pltpu.sync_copy(s1_v, xs1_sh.at[pl.ds(base, SLICE)])
    pltpu.sync_copy(s0_v, out0_sh.at[pl.ds(base, SLICE)])
    pltpu.sync_copy(s1_v, out1_sh.at[pl.ds(base, SLICE)])
    plsc.subcore_barrier()

    pltpu.sync_copy(xs0_sh, x0_v)
    pltpu.sync_copy(xs1_sh, x1_v)

    def edge_step(j, carry):
        for k in range(CHUNK // 16):
            src = src_v[j, pl.ds(k * 16, 16)]
            vals0_v[pl.ds(k * 16, 16)] = plsc.load_gather(x0_v, [src])
            vals1_v[pl.ds(k * 16, 16)] = plsc.load_gather(x1_v, [src])
        pltpu.sync_copy(vals0_v, out0_sh.at[dst_v.at[j]], add=True)
        pltpu.sync_copy(vals1_v, out1_sh.at[dst_v.at[j]], add=True)
        return carry
    lax.fori_loop(0, nch, edge_step, 0)
    plsc.subcore_barrier()

    pltpu.sync_copy(out0_sh.at[pl.ds(base, SLICE)], s0_v)
    pltpu.sync_copy(out1_sh.at[pl.ds(base, SLICE)], s1_v)
    for i in range(SLICE // 16):
        y = dinvs_v[pl.ds(i * 16, 16)]
        s0_v[pl.ds(i * 16, 16)] = s0_v[pl.ds(i * 16, 16)] * y
        s1_v[pl.ds(i * 16, 16)] = s1_v[pl.ds(i * 16, 16)] * y
    pltpu.sync_copy(s0_v, out0_hbm.at[pl.ds(c * NN_PAD + base, SLICE)])
    pltpu.sync_copy(s1_v, out1_hbm.at[pl.ds(c * NN_PAD + base, SLICE)])


@functools.cache
def _sc_scatter_kernel():
    return functools.partial(
        pl.kernel,
        out_type=[jax.ShapeDtypeStruct((2 * NN_PAD,), jnp.float32),
                  jax.ShapeDtypeStruct((2 * NN_PAD,), jnp.float32)],
        mesh=plsc.VectorSubcoreMesh(core_axis_name="c", subcore_axis_name="s",
                                    num_cores=2, num_subcores=16),
        compiler_params=pltpu.CompilerParams(needs_layout_passes=False),
        scratch_types=[
        pltpu.VMEM((NCHUNK, CHUNK), jnp.int32),
        pltpu.VMEM((NCHUNK, CHUNK), jnp.int32),
        pltpu.VMEM((NN_PAD,), jnp.float32),
        pltpu.VMEM((NN_PAD,), jnp.float32),
        pltpu.VMEM((CHUNK,), jnp.float32),
        pltpu.VMEM((CHUNK,), jnp.float32),
        pltpu.VMEM((SLICE,), jnp.float32),
        pltpu.VMEM((SLICE,), jnp.float32),
        pltpu.VMEM((SLICE,), jnp.float32),
        pltpu.VMEM((SLICE,), jnp.float32),
        pltpu.VMEM((CHUNK,), jnp.float32),
        pltpu.VMEM((TBL,), jnp.float32),
        pltpu.VMEM_SHARED((NN_PAD,), jnp.float32),
        pltpu.VMEM_SHARED((NN_PAD,), jnp.float32),
        pltpu.VMEM_SHARED((NN_PAD,), jnp.float32),
        pltpu.VMEM_SHARED((NN_PAD,), jnp.float32),
        pltpu.VMEM_SHARED((NN_PAD,), jnp.float32),
        ],
    )(_sc_body)


def _fold_body(p_ref, w_ref, u_ref):
    u_ref[...] = jnp.dot(p_ref[...], w_ref[...],
                         preferred_element_type=jnp.float32)


def _fold_u(P, W_ih):
    n_blk = 8
    blk = (4 * HID) // n_blk
    return pl.pallas_call(
        _fold_body,
        grid=(n_blk,),
        in_specs=[
            pl.BlockSpec((72, GCN_OUT * NUM_NODES), lambda n: (0, 0)),
            pl.BlockSpec((GCN_OUT * NUM_NODES, blk), lambda n: (0, n)),
        ],
        out_specs=pl.BlockSpec((72, blk), lambda n: (0, n)),
        out_shape=jax.ShapeDtypeStruct((72, 4 * HID), jnp.float32),
    )(P, W_ih)


def _seq_body(o0_ref, o1_ref, u_ref, whh_ref, bsum_ref,
              w1_ref, b1_ref, w2_ref, b2_ref, w3_ref, b3_ref,
              w4_ref, b4_ref, sx_ref, sy_ref, stats_ref, preds_ref):
    U0 = u_ref[0:32, :]
    U1 = u_ref[32:64, :]
    bvec = u_ref[64:65, :]
    bias = bvec + bsum_ref[...]
    std0 = stats_ref[0:1, 0:1]
    std1 = stats_ref[0:1, 1:2]
    mean0 = stats_ref[1:2, 0:1]
    mean1 = stats_ref[1:2, 1:2]

    h = jnp.zeros((T, HID), jnp.float32)
    c = jnp.zeros((T, HID), jnp.float32)
    p = None
    for step in range(S_IN - 1 + S_OUT):
        if step < S_IN:
            m0 = o0_ref[step]
            m1 = o1_ref[step]
        else:
            pxs = jnp.dot(p, sx_ref[...], preferred_element_type=jnp.float32)
            pys = jnp.dot(p, sy_ref[...], preferred_element_type=jnp.float32)
            fx = pxs * std0 + mean0
            fy = pys * std1 + mean1
            col = lax.broadcasted_iota(jnp.int32, (T, NODE_PAD), 1)
            exn = jnp.where((fx > 0.04) & (fy > 0.04) & (col < NUM_NODES),
                            1.0, 0.0)
            dx = fx[:, :, None] - fx[:, None, :]
            dy = fy[:, :, None] - fy[:, None, :]
            d2 = dx * dx + dy * dy
            cond = jnp.where((d2 > 0.0) & (d2 < 100.0), 1.0, 0.0)
            ep = exn[:, :, None] * exn[:, None, :]
            r = lax.broadcasted_iota(jnp.int32, (T, NODE_PAD, NODE_PAD), 1)
            q = lax.broadcasted_iota(jnp.int32, (T, NODE_PAD, NODE_PAD), 2)
            eye = r == q
            A = jnp.where(eye, 1.0, ep * cond)
            deg = jnp.sum(A, axis=-1)
            dinv = lax.rsqrt(jnp.maximum(deg, 1e-12))
            An = A * (dinv[:, :, None] * dinv[:, None, :])
            m0 = jnp.sum(An * pxs[:, None, :], axis=-1)
            m1 = jnp.sum(An * pys[:, None, :], axis=-1)
        gates = (jnp.dot(m0, U0, preferred_element_type=jnp.float32)
                 + jnp.dot(m1, U1, preferred_element_type=jnp.float32)
                 + jnp.dot(h, whh_ref[...], preferred_element_type=jnp.float32)
                 + bias)
        i_g = jax.nn.sigmoid(gates[:, 0:HID])
        f_g = jax.nn.sigmoid(gates[:, HID:2 * HID])
        g_g = jnp.tanh(gates[:, 2 * HID:3 * HID])
        o_g = jax.nn.sigmoid(gates[:, 3 * HID:4 * HID])
        c = f_g * c + i_g * g_g
        h = o_g * jnp.tanh(c)
        m = jax.nn.relu(jnp.dot(h, w1_ref[...],
                                preferred_element_type=jnp.float32)
                        + b1_ref[...])
        m = jax.nn.relu(jnp.dot(m, w2_ref[...],
                                preferred_element_type=jnp.float32)
                        + b2_ref[...])
        m = jax.nn.relu(jnp.dot(m, w3_ref[...],
                                preferred_element_type=jnp.float32)
                        + b3_ref[...])
        p = jnp.dot(m, w4_ref[...],
                    preferred_element_type=jnp.float32) + b4_ref[...]
        preds_ref[step] = p


def _seq_run(o0p, o1p, U, W_hh, bsum, W1, b1, W2, b2, W3, b3, W4, b4,
             Sx, Sy, stats):
    return pl.pallas_call(
        _seq_body,
        out_shape=jax.ShapeDtypeStruct((7, T, NUM_IN * NUM_NODES),
                                       jnp.float32),
    )(o0p, o1p, U, W_hh, bsum, W1, b1, W2, b2, W3, b3, W4, b4, Sx, Sy, stats)


def kernel(feature_input, edge_index, batch_index, number_of_trajectories,
           stats, gcn_W, gcn_b, W_ih, W_hh, b_ih, b_hh,
           W1, b1, W2, b2, W3, b3, W4, b4):
    ei = edge_index.astype(jnp.int32)
    src0 = jnp.concatenate([ei[0, 0], ei[1, 0] + N_TOTAL, ei[2, 0] + 2 * N_TOTAL])
    dst0 = jnp.concatenate([ei[0, 1], ei[1, 1] + N_TOTAL, ei[2, 1] + 2 * N_TOTAL])
    def _tile_pad(arr, fill):
        real = arr.reshape(16, 4096)
        dummy = jnp.full((16, E_TILE - 4096), fill, jnp.int32)
        return jnp.concatenate([real, dummy], axis=1).reshape(-1)
    src1 = _tile_pad(jnp.concatenate([ei[3, 0], ei[4, 0] + N_TOTAL]), 0)
    dst1 = _tile_pad(jnp.concatenate([ei[3, 1], ei[4, 1] + N_TOTAL]),
                     DUMMY_DST)
    src_e = jnp.stack([src0, src1]).reshape(2, E_CORE // CHUNK, CHUNK)
    dst_e = jnp.stack([dst0, dst1]).reshape(2, E_CORE // CHUNK, CHUNK)

    xf = feature_input.reshape(S_IN * N_TOTAL, NUM_IN)
    pad0 = NN_PAD - 3 * N_TOTAL
    pad1 = NN_PAD - 2 * N_TOTAL
    x0_in = jnp.concatenate([
        jnp.pad(xf[:3 * N_TOTAL, 0], (0, pad0)),
        jnp.pad(xf[3 * N_TOTAL:, 0], (0, pad1)),
    ])
    x1_in = jnp.concatenate([
        jnp.pad(xf[:3 * N_TOTAL, 1], (0, pad0)),
        jnp.pad(xf[3 * N_TOTAL:, 1], (0, pad1)),
    ])

    rsqrt_tbl = lax.rsqrt(jnp.maximum(
        jnp.arange(TBL, dtype=jnp.float32), 1.0))
    out0, out1 = _sc_scatter_kernel()(src_e, dst_e, x0_in, x1_in, rsqrt_tbl)
    o0 = jnp.concatenate([out0[:3 * N_TOTAL],
                          out0[NN_PAD:NN_PAD + 2 * N_TOTAL]])
    o1 = jnp.concatenate([out1[:3 * N_TOTAL],
                          out1[NN_PAD:NN_PAD + 2 * N_TOTAL]])
    o0p = jnp.pad(o0.reshape(S_IN, T, NUM_NODES), ((0, 0), (0, 0), (0, 2)))
    o1p = jnp.pad(o1.reshape(S_IN, T, NUM_NODES), ((0, 0), (0, 0), (0, 2)))

    K = GCN_OUT * NUM_NODES
    eye30 = jnp.eye(NUM_NODES, dtype=jnp.float32)
    P0 = jnp.kron(eye30, gcn_W[0:1, :])
    P1 = jnp.kron(eye30, gcn_W[1:2, :])
    bb = jnp.tile(gcn_b, NUM_NODES)[None, :]
    zrow2 = jnp.zeros((2, K), jnp.float32)
    zrow7 = jnp.zeros((7, K), jnp.float32)
    P = jnp.concatenate([P0, zrow2, P1, zrow2, bb, zrow7])
    U = _fold_u(P, W_ih)

    bsum = (b_ih + b_hh)[None, :]
    k60 = jnp.arange(NUM_IN * NUM_NODES)[:, None]
    n32 = jnp.arange(NODE_PAD)[None, :]
    Sx = ((k60 == 2 * n32) & (n32 < NUM_NODES)).astype(jnp.float32)
    Sy = ((k60 == 2 * n32 + 1) & (n32 < NUM_NODES)).astype(jnp.float32)

    preds = _seq_run(o0p, o1p, U, W_hh, bsum,
                     W1, b1[None, :], W2, b2[None, :], W3, b3[None, :],
                     W4, b4[None, :], Sx, Sy, stats)

    enc = jnp.concatenate([
        feature_input[0][None],
        preds[:S_IN - 1].reshape(S_IN - 1, N_TOTAL, NUM_IN),
    ])
    dec = preds[S_IN - 1:].reshape(S_OUT, N_TOTAL, NUM_IN)
    return enc, dec


# speedup vs baseline: 29.8690x; 1.0782x over previous
import functools

import jax
import jax.numpy as jnp
from jax import lax
from jax.experimental import pallas as pl
from jax.experimental.pallas import tpu as pltpu
from jax.experimental.pallas import tpu_sc as plsc

NUM_NODES = 30
NUM_IN = 2
GCN_OUT = 64
HID = 512
T = 128
S_IN = 5
S_OUT = 3
N_TOTAL = T * NUM_NODES
N_EDGES = 32768

NN_PAD = 11776
SLICE = NN_PAD // 16
E_CORE = 3 * N_EDGES
E_TILE = E_CORE // 16
CHUNK = 128
NCHUNK = E_TILE // CHUNK
DUMMY_DST = 11520
NODE_PAD = 32


TBL = 32776


def _sc_body(src_hbm, dst_hbm, x0_hbm, x1_hbm, tbl_hbm, out0_hbm, out1_hbm,
             src_v, dst_v, x0_v, x1_v, vals0_v, vals1_v, vals0b_v, vals1b_v,
             degs_v, dinvs_v, s0_v, s1_v, ones_v, table_v,
             sem_a0, sem_a1, sem_b0, sem_b1,
             deg_sh, xs0_sh, xs1_sh, out0_sh, out1_sh):
    c = lax.axis_index("c")
    s = lax.axis_index("s")
    base = s * SLICE
    nch = jnp.where(c == 0, NCHUNK, (2 * N_EDGES) // (16 * CHUNK))

    pltpu.sync_copy(src_hbm.at[c, pl.ds(s * NCHUNK, NCHUNK)], src_v)
    pltpu.sync_copy(dst_hbm.at[c, pl.ds(s * NCHUNK, NCHUNK)], dst_v)
    pltpu.sync_copy(x0_hbm.at[pl.ds(c * NN_PAD + base, SLICE)], s0_v)
    pltpu.sync_copy(x1_hbm.at[pl.ds(c * NN_PAD + base, SLICE)], s1_v)
    pltpu.sync_copy(tbl_hbm, table_v)

    for i in range(CHUNK // 16):
        ones_v[pl.ds(i * 16, 16)] = jnp.full((16,), 1.0, jnp.float32)
    for i in range(SLICE // 16):
        degs_v[pl.ds(i * 16, 16)] = jnp.full((16,), 0.0, jnp.float32)
    pltpu.sync_copy(degs_v, deg_sh.at[pl.ds(base, SLICE)])
    plsc.subcore_barrier()

    def deg_step(j, carry):
        @pl.when(j % 2 == 0)
        def _even():
            @pl.when(j >= 2)
            def _w():
                pltpu.make_async_copy(ones_v, deg_sh.at[dst_v.at[j]],
                                      sem_a0).wait()
            pltpu.async_copy(ones_v, deg_sh.at[dst_v.at[j]], sem_a0,
                             add=True)

        @pl.when(j % 2 == 1)
        def _odd():
            @pl.when(j >= 2)
            def _w():
                pltpu.make_async_copy(ones_v, deg_sh.at[dst_v.at[j]],
                                      sem_b0).wait()
            pltpu.async_copy(ones_v, deg_sh.at[dst_v.at[j]], sem_b0,
                             add=True)
        return carry
    lax.fori_loop(0, nch, deg_step, 0)
    pltpu.make_async_copy(ones_v, deg_sh.at[dst_v.at[0]], sem_a0).wait()
    pltpu.make_async_copy(ones_v, deg_sh.at[dst_v.at[0]], sem_b0).wait()
    plsc.subcore_barrier()

    pltpu.sync_copy(deg_sh.at[pl.ds(base, SLICE)], degs_v)
    for i in range(SLICE // 16):
        d = degs_v[pl.ds(i * 16, 16)] + 1.0
        y = plsc.load_gather(table_v, [d.astype(jnp.int32)])
        dinvs_v[pl.ds(i * 16, 16)] = y
        s0_v[pl.ds(i * 16, 16)] = s0_v[pl.ds(i * 16, 16)] * y
        s1_v[pl.ds(i * 16, 16)] = s1_v[pl.ds(i * 16, 16)] * y
    pltpu.sync_copy(s0_v, xs0_sh.at[pl.ds(base, SLICE)])
    pltpu.sync_copy(s1_v, xs1_sh.at[pl.ds(base, SLICE)])
    pltpu.sync_copy(s0_v, out0_sh.at[pl.ds(base, SLICE)])
    pltpu.sync_copy(s1_v, out1_sh.at[pl.ds(base, SLICE)])
    plsc.subcore_barrier()

    pltpu.sync_copy(xs0_sh, x0_v)
    pltpu.sync_copy(xs1_sh, x1_v)

    def edge_step(j, carry):
        def run(b0, b1, s0, s1):
            @pl.when(j >= 2)
            def _w():
                pltpu.make_async_copy(b0, out0_sh.at[dst_v.at[j]], s0).wait()
                pltpu.make_async_copy(b1, out1_sh.at[dst_v.at[j]], s1).wait()
            for k in range(CHUNK // 16):
                src = src_v[j, pl.ds(k * 16, 16)]
                b0[pl.ds(k * 16, 16)] = plsc.load_gather(x0_v, [src])
                b1[pl.ds(k * 16, 16)] = plsc.load_gather(x1_v, [src])
            pltpu.async_copy(b0, out0_sh.at[dst_v.at[j]], s0, add=True)
            pltpu.async_copy(b1, out1_sh.at[dst_v.at[j]], s1, add=True)

        @pl.when(j % 2 == 0)
        def _even():
            run(vals0_v, vals1_v, sem_a0, sem_a1)

        @pl.when(j % 2 == 1)
        def _odd():
            run(vals0b_v, vals1b_v, sem_b0, sem_b1)
        return carry
    lax.fori_loop(0, nch, edge_step, 0)
    pltpu.make_async_copy(vals0_v, out0_sh.at[dst_v.at[0]], sem_a0).wait()
    pltpu.make_async_copy(vals1_v, out1_sh.at[dst_v.at[0]], sem_a1).wait()
    pltpu.make_async_copy(vals0b_v, out0_sh.at[dst_v.at[0]], sem_b0).wait()
    pltpu.make_async_copy(vals1b_v, out1_sh.at[dst_v.at[0]], sem_b1).wait()
    plsc.subcore_barrier()

    pltpu.sync_copy(out0_sh.at[pl.ds(base, SLICE)], s0_v)
    pltpu.sync_copy(out1_sh.at[pl.ds(base, SLICE)], s1_v)
    for i in range(SLICE // 16):
        y = dinvs_v[pl.ds(i * 16, 16)]
        s0_v[pl.ds(i * 16, 16)] = s0_v[pl.ds(i * 16, 16)] * y
        s1_v[pl.ds(i * 16, 16)] = s1_v[pl.ds(i * 16, 16)] * y
    pltpu.sync_copy(s0_v, out0_hbm.at[pl.ds(c * NN_PAD + base, SLICE)])
    pltpu.sync_copy(s1_v, out1_hbm.at[pl.ds(c * NN_PAD + base, SLICE)])


@functools.cache
def _sc_scatter_kernel():
    return functools.partial(
        pl.kernel,
        out_type=[jax.ShapeDtypeStruct((2 * NN_PAD,), jnp.float32),
                  jax.ShapeDtypeStruct((2 * NN_PAD,), jnp.float32)],
        mesh=plsc.VectorSubcoreMesh(core_axis_name="c", subcore_axis_name="s",
                                    num_cores=2, num_subcores=16),
        compiler_params=pltpu.CompilerParams(needs_layout_passes=False),
        scratch_types=[
        pltpu.VMEM((NCHUNK, CHUNK), jnp.int32),
        pltpu.VMEM((NCHUNK, CHUNK), jnp.int32),
        pltpu.VMEM((NN_PAD,), jnp.float32),
        pltpu.VMEM((NN_PAD,), jnp.float32),
        pltpu.VMEM((CHUNK,), jnp.float32),
        pltpu.VMEM((CHUNK,), jnp.float32),
        pltpu.VMEM((CHUNK,), jnp.float32),
        pltpu.VMEM((CHUNK,), jnp.float32),
        pltpu.VMEM((SLICE,), jnp.float32),
        pltpu.VMEM((SLICE,), jnp.float32),
        pltpu.VMEM((SLICE,), jnp.float32),
        pltpu.VMEM((SLICE,), jnp.float32),
        pltpu.VMEM((CHUNK,), jnp.float32),
        pltpu.VMEM((TBL,), jnp.float32),
        pltpu.SemaphoreType.DMA,
        pltpu.SemaphoreType.DMA,
        pltpu.SemaphoreType.DMA,
        pltpu.SemaphoreType.DMA,
        pltpu.VMEM_SHARED((NN_PAD,), jnp.float32),
        pltpu.VMEM_SHARED((NN_PAD,), jnp.float32),
        pltpu.VMEM_SHARED((NN_PAD,), jnp.float32),
        pltpu.VMEM_SHARED((NN_PAD,), jnp.float32),
        pltpu.VMEM_SHARED((NN_PAD,), jnp.float32),
        ],
    )(_sc_body)


def _fold_body(p_ref, w_ref, u_ref):
    u_ref[...] = jnp.dot(p_ref[...], w_ref[...],
                         preferred_element_type=jnp.float32)


def _fold_u(P, W_ih):
    n_blk = 8
    blk = (4 * HID) // n_blk
    return pl.pallas_call(
        _fold_body,
        grid=(n_blk,),
        in_specs=[
            pl.BlockSpec((72, GCN_OUT * NUM_NODES), lambda n: (0, 0)),
            pl.BlockSpec((GCN_OUT * NUM_NODES, blk), lambda n: (0, n)),
        ],
        out_specs=pl.BlockSpec((72, blk), lambda n: (0, n)),
        out_shape=jax.ShapeDtypeStruct((72, 4 * HID), jnp.float32),
    )(P, W_ih)


def _seq_body(o0_ref, o1_ref, u_ref, whh_ref, bsum_ref,
              w1_ref, b1_ref, w2_ref, b2_ref, w3_ref, b3_ref,
              w4_ref, b4_ref, sx_ref, sy_ref, stats_ref, preds_ref):
    U0 = u_ref[0:32, :]
    U1 = u_ref[32:64, :]
    bvec = u_ref[64:65, :]
    bias = bvec + bsum_ref[...]
    std0 = stats_ref[0:1, 0:1]
    std1 = stats_ref[0:1, 1:2]
    mean0 = stats_ref[1:2, 0:1]
    mean1 = stats_ref[1:2, 1:2]

    h = jnp.zeros((T, HID), jnp.float32)
    c = jnp.zeros((T, HID), jnp.float32)
    p = None
    for step in range(S_IN - 1 + S_OUT):
        if step < S_IN:
            m0 = o0_ref[step]
            m1 = o1_ref[step]
        else:
            pxs = jnp.dot(p, sx_ref[...], preferred_element_type=jnp.float32)
            pys = jnp.dot(p, sy_ref[...], preferred_element_type=jnp.float32)
            fx = pxs * std0 + mean0
            fy = pys * std1 + mean1
            col = lax.broadcasted_iota(jnp.int32, (T, NODE_PAD), 1)
            exn = jnp.where((fx > 0.04) & (fy > 0.04) & (col < NUM_NODES),
                            1.0, 0.0)
            dx = fx[:, :, None] - fx[:, None, :]
            dy = fy[:, :, None] - fy[:, None, :]
            d2 = dx * dx + dy * dy
            cond = jnp.where((d2 > 0.0) & (d2 < 100.0), 1.0, 0.0)
            ep = exn[:, :, None] * exn[:, None, :]
            r = lax.broadcasted_iota(jnp.int32, (T, NODE_PAD, NODE_PAD), 1)
            q = lax.broadcasted_iota(jnp.int32, (T, NODE_PAD, NODE_PAD), 2)
            eye = r == q
            A = jnp.where(eye, 1.0, ep * cond)
            deg = jnp.sum(A, axis=-1)
            dinv = lax.rsqrt(jnp.maximum(deg, 1e-12))
            An = A * (dinv[:, :, None] * dinv[:, None, :])
            m0 = jnp.sum(An * pxs[:, None, :], axis=-1)
            m1 = jnp.sum(An * pys[:, None, :], axis=-1)
        gates = (jnp.dot(m0, U0, preferred_element_type=jnp.float32)
                 + jnp.dot(m1, U1, preferred_element_type=jnp.float32)
                 + jnp.dot(h, whh_ref[...], preferred_element_type=jnp.float32)
                 + bias)
        i_g = jax.nn.sigmoid(gates[:, 0:HID])
        f_g = jax.nn.sigmoid(gates[:, HID:2 * HID])
        g_g = jnp.tanh(gates[:, 2 * HID:3 * HID])
        o_g = jax.nn.sigmoid(gates[:, 3 * HID:4 * HID])
        c = f_g * c + i_g * g_g
        h = o_g * jnp.tanh(c)
        m = jax.nn.relu(jnp.dot(h, w1_ref[...],
                                preferred_element_type=jnp.float32)
                        + b1_ref[...])
        m = jax.nn.relu(jnp.dot(m, w2_ref[...],
                                preferred_element_type=jnp.float32)
                        + b2_ref[...])
        m = jax.nn.relu(jnp.dot(m, w3_ref[...],
                                preferred_element_type=jnp.float32)
                        + b3_ref[...])
        p = jnp.dot(m, w4_ref[...],
                    preferred_element_type=jnp.float32) + b4_ref[...]
        preds_ref[step] = p


def _seq_run(o0p, o1p, U, W_hh, bsum, W1, b1, W2, b2, W3, b3, W4, b4,
             Sx, Sy, stats):
    return pl.pallas_call(
        _seq_body,
        out_shape=jax.ShapeDtypeStruct((7, T, NUM_IN * NUM_NODES),
                                       jnp.float32),
    )(o0p, o1p, U, W_hh, bsum, W1, b1, W2, b2, W3, b3, W4, b4, Sx, Sy, stats)


def kernel(feature_input, edge_index, batch_index, number_of_trajectories,
           stats, gcn_W, gcn_b, W_ih, W_hh, b_ih, b_hh,
           W1, b1, W2, b2, W3, b3, W4, b4):
    ei = edge_index.astype(jnp.int32)
    src0 = jnp.concatenate([ei[0, 0], ei[1, 0] + N_TOTAL, ei[2, 0] + 2 * N_TOTAL])
    dst0 = jnp.concatenate([ei[0, 1], ei[1, 1] + N_TOTAL, ei[2, 1] + 2 * N_TOTAL])
    def _tile_pad(arr, fill):
        real = arr.reshape(16, 4096)
        dummy = jnp.full((16, E_TILE - 4096), fill, jnp.int32)
        return jnp.concatenate([real, dummy], axis=1).reshape(-1)
    src1 = _tile_pad(jnp.concatenate([ei[3, 0], ei[4, 0] + N_TOTAL]), 0)
    dst1 = _tile_pad(jnp.concatenate([ei[3, 1], ei[4, 1] + N_TOTAL]),
                     DUMMY_DST)
    src_e = jnp.stack([src0, src1]).reshape(2, E_CORE // CHUNK, CHUNK)
    dst_e = jnp.stack([dst0, dst1]).reshape(2, E_CORE // CHUNK, CHUNK)

    xf = feature_input.reshape(S_IN * N_TOTAL, NUM_IN)
    pad0 = NN_PAD - 3 * N_TOTAL
    pad1 = NN_PAD - 2 * N_TOTAL
    x0_in = jnp.concatenate([
        jnp.pad(xf[:3 * N_TOTAL, 0], (0, pad0)),
        jnp.pad(xf[3 * N_TOTAL:, 0], (0, pad1)),
    ])
    x1_in = jnp.concatenate([
        jnp.pad(xf[:3 * N_TOTAL, 1], (0, pad0)),
        jnp.pad(xf[3 * N_TOTAL:, 1], (0, pad1)),
    ])

    rsqrt_tbl = lax.rsqrt(jnp.maximum(
        jnp.arange(TBL, dtype=jnp.float32), 1.0))
    out0, out1 = _sc_scatter_kernel()(src_e, dst_e, x0_in, x1_in, rsqrt_tbl)
    o0 = jnp.concatenate([out0[:3 * N_TOTAL],
                          out0[NN_PAD:NN_PAD + 2 * N_TOTAL]])
    o1 = jnp.concatenate([out1[:3 * N_TOTAL],
                          out1[NN_PAD:NN_PAD + 2 * N_TOTAL]])
    o0p = jnp.pad(o0.reshape(S_IN, T, NUM_NODES), ((0, 0), (0, 0), (0, 2)))
    o1p = jnp.pad(o1.reshape(S_IN, T, NUM_NODES), ((0, 0), (0, 0), (0, 2)))

    K = GCN_OUT * NUM_NODES
    eye30 = jnp.eye(NUM_NODES, dtype=jnp.float32)
    P0 = jnp.kron(eye30, gcn_W[0:1, :])
    P1 = jnp.kron(eye30, gcn_W[1:2, :])
    bb = jnp.tile(gcn_b, NUM_NODES)[None, :]
    zrow2 = jnp.zeros((2, K), jnp.float32)
    zrow7 = jnp.zeros((7, K), jnp.float32)
    P = jnp.concatenate([P0, zrow2, P1, zrow2, bb, zrow7])
    U = _fold_u(P, W_ih)

    bsum = (b_ih + b_hh)[None, :]
    k60 = jnp.arange(NUM_IN * NUM_NODES)[:, None]
    n32 = jnp.arange(NODE_PAD)[None, :]
    Sx = ((k60 == 2 * n32) & (n32 < NUM_NODES)).astype(jnp.float32)
    Sy = ((k60 == 2 * n32 + 1) & (n32 < NUM_NODES)).astype(jnp.float32)

    preds = _seq_run(o0p, o1p, U, W_hh, bsum,
                     W1, b1[None, :], W2, b2[None, :], W3, b3[None, :],
                     W4, b4[None, :], Sx, Sy, stats)

    enc = jnp.concatenate([
        feature_input[0][None],
        preds[:S_IN - 1].reshape(S_IN - 1, N_TOTAL, NUM_IN),
    ])
    dec = preds[S_IN - 1:].reshape(S_OUT, N_TOTAL, NUM_IN)
    return enc, dec
